# baseline (speedup 1.0000x reference)
import functools

import jax
import jax.numpy as jnp
from jax import lax
from jax.experimental import pallas as pl
from jax.experimental.pallas import tpu as pltpu
from jax.experimental.pallas import tpu_sc as plsc

B = 16384
DEG = 32
D = 128
NC = 2
NS = 16
NW = NC * NS
BPW = B // NW
CH = 128
NPC = CH // DEG
NCHUNK = BPW * DEG // CH
TCB = 1024
NTB = B // TCB


def _sc_gather(table, neigh_flat, nodes):
    mesh = plsc.VectorSubcoreMesh(core_axis_name="c", subcore_axis_name="s")

    @functools.partial(
        pl.kernel,
        mesh=mesh,
        compiler_params=pltpu.CompilerParams(use_tc_tiling_on_sc=False,
                                             needs_layout_passes=False),
        out_type=[
            jax.ShapeDtypeStruct((B, D), jnp.float32),
            jax.ShapeDtypeStruct((B, D), jnp.float32),
        ],
        scratch_types=[
            pltpu.VMEM((BPW * DEG,), jnp.int32),
            pltpu.VMEM((BPW,), jnp.int32),
            pltpu.VMEM((2, CH, D), jnp.float32),
            pltpu.VMEM((BPW, D), jnp.float32),
            pltpu.SemaphoreType.DMA((2,)),
        ],
    )
    def k(table_h, gidx_h, nidx_h, self_o, nsum_o, gidx, nidx, bufs, acc,
          gsem):
        c = lax.axis_index("c")
        s = lax.axis_index("s")
        base = (c * NS + s) * BPW

        pltpu.sync_copy(gidx_h.at[pl.ds(base * DEG, BPW * DEG)], gidx)
        pltpu.sync_copy(nidx_h.at[pl.ds(base, BPW)], nidx)

        def gcopy(ci, b):
            off = pl.multiple_of(ci * CH, CH)
            return pltpu.make_async_copy(
                table_h.at[gidx.at[pl.ds(off, CH)]], bufs.at[b], gsem.at[b])

        def reduce_chunk(ci, b):
            def nbody(n, carry):
                sums = [jnp.zeros((16,), jnp.float32) for _ in range(8)]
                for r in range(DEG):
                    q = n * DEG + r
                    for g in range(8):
                        sums[g] = sums[g] + bufs[b, q, pl.ds(g * 16, 16)]
                row = ci * NPC + n
                for g in range(8):
                    acc[row, pl.ds(g * 16, 16)] = sums[g]
                return carry

            lax.fori_loop(0, NPC, nbody, 0)

        gcopy(0, 0).start()

        def body(i, carry):
            c0 = 2 * i
            gcopy(c0 + 1, 1).start()
            gcopy(c0, 0).wait()
            reduce_chunk(c0, 0)

            @pl.when(i < NCHUNK // 2 - 1)
            def _():
                gcopy(c0 + 2, 0).start()

            gcopy(c0 + 1, 1).wait()
            reduce_chunk(c0 + 1, 1)
            return carry

        lax.fori_loop(0, NCHUNK // 2, body, 0)

        def sget(kk, b):
            return pltpu.make_async_copy(
                table_h.at[nidx.at[pl.ds(kk * CH, CH)]], bufs.at[b],
                gsem.at[b])

        for kk in range(0, BPW // CH, 2):
            sget(kk, 0).start()
            sget(kk + 1, 1).start()
            for b in range(2):
                sget(kk + b, b).wait()
                dst = pl.multiple_of(base + (kk + b) * CH, CH)
                pltpu.sync_copy(bufs.at[b], self_o.at[pl.ds(dst, CH)])

        pltpu.sync_copy(acc, nsum_o.at[pl.ds(pl.multiple_of(base, CH), BPW)])

    return k(table, neigh_flat, nodes)


def _tc_dense(self_feats, nsum, W1, b1, gamma, beta):
    def body(x_ref, n_ref, w_ref, b_ref, g_ref, bb_ref, o_ref,
             lin_ref, ps_ref):
        p = pl.program_id(0)
        i = pl.program_id(1)

        @pl.when(p == 0)
        def _():
            x = x_ref[...]
            n = n_ref[...] * (1.0 / DEG)
            w = w_ref[...]
            dn = (((1,), (1,)), ((), ()))
            lin = lax.dot_general(x, w[:, :D], dn,
                                  preferred_element_type=jnp.float32)
            lin = lin + lax.dot_general(n, w[:, D:], dn,
                                        preferred_element_type=jnp.float32)
            lin = lin + b_ref[...]
            lin_ref[pl.ds(i * TCB, TCB), :] = lin
            s1 = jnp.sum(lin, axis=0, keepdims=True)
            s2 = jnp.sum(lin * lin, axis=0, keepdims=True)
            ps = jnp.concatenate([s1, s2], axis=0)

            @pl.when(i == 0)
            def _():
                ps_ref[...] = ps

            @pl.when(i > 0)
            def _():
                ps_ref[...] = ps_ref[...] + ps

        @pl.when(p == 1)
        def _():
            ps = ps_ref[...]
            mu = ps[0:1, :] * (1.0 / B)
            var = ps[1:2, :] * (1.0 / B) - mu * mu
            scale = lax.rsqrt(var + 1e-5) * g_ref[...]
            lb = lin_ref[pl.ds(i * TCB, TCB), :]
            o_ref[...] = jnp.maximum((lb - mu) * scale + bb_ref[...], 0.0)

    return pl.pallas_call(
        body,
        grid=(2, NTB),
        in_specs=[
            pl.BlockSpec((TCB, D), lambda p, i: (i * (1 - p), 0)),
            pl.BlockSpec((TCB, D), lambda p, i: (i * (1 - p), 0)),
            pl.BlockSpec((D, 2 * D), lambda p, i: (0, 0)),
            pl.BlockSpec((1, D), lambda p, i: (0, 0)),
            pl.BlockSpec((1, D), lambda p, i: (0, 0)),
            pl.BlockSpec((1, D), lambda p, i: (0, 0)),
        ],
        out_specs=pl.BlockSpec((TCB, D), lambda p, i: (i, 0)),
        out_shape=jax.ShapeDtypeStruct((B, D), jnp.float32),
        scratch_shapes=[
            pltpu.VMEM((B, D), jnp.float32),
            pltpu.VMEM((2, D), jnp.float32),
        ],
    )(self_feats, nsum, W1,
      b1.reshape(1, D), gamma.reshape(1, D), beta.reshape(1, D))


def kernel(nodes, neighbors, emb_table, W1, b1, gamma, beta):
    self_feats, nsum = _sc_gather(emb_table, neighbors.reshape(-1), nodes)
    return _tc_dense(self_feats, nsum, W1, b1, gamma, beta)


# speedup vs baseline: 1.0050x; 1.0050x over previous
import functools

import jax
import jax.numpy as jnp
from jax import lax
from jax.experimental import pallas as pl
from jax.experimental.pallas import tpu as pltpu
from jax.experimental.pallas import tpu_sc as plsc

B = 16384
DEG = 32
D = 128
NC = 2
NS = 16
NW = NC * NS
BPW = B // NW
CH = 128
NPC = CH // DEG
NCHUNK = BPW * DEG // CH
NBUF = 4
HPW = BPW // 2
NGRP = NCHUNK // NBUF


def _sc_gather(table, neigh_flat, nodes, zeros):
    mesh = plsc.VectorSubcoreMesh(core_axis_name="c", subcore_axis_name="s")

    @functools.partial(
        pl.kernel,
        mesh=mesh,
        out_type=[
            jax.ShapeDtypeStruct((B, D), jnp.float32),
            jax.ShapeDtypeStruct((B, D), jnp.float32),
        ],
        scratch_types=[
            pltpu.VMEM((BPW * DEG,), jnp.int32),
            pltpu.VMEM((BPW,), jnp.int32),
            pltpu.VMEM((NBUF, CH, D), jnp.float32),
            pltpu.VMEM((NBUF, CH), jnp.int32),
            pltpu.VMEM_SHARED((NS * HPW, D), jnp.float32),
            pltpu.SemaphoreType.DMA((NBUF,)),
            pltpu.SemaphoreType.DMA((NBUF,)),
            pltpu.SemaphoreType.DMA,
        ],
    )
    def k(table_h, gidx_h, nidx_h, zeros_h, self_o, nsum_o,
          gidx, nidx, bufs, scat, acc, gsem, ssem, zsem):
        c = lax.axis_index("c")
        s = lax.axis_index("s")
        base = (c * NS + s) * BPW

        pltpu.async_copy(zeros_h, acc.at[pl.ds(s * HPW, HPW)], zsem)
        pltpu.sync_copy(gidx_h.at[pl.ds(base * DEG, BPW * DEG)], gidx)
        pltpu.sync_copy(nidx_h.at[pl.ds(base, BPW)], nidx)

        def gcopy(ci, b):
            off = pl.multiple_of(ci * CH, CH)
            return pltpu.make_async_copy(
                table_h.at[gidx.at[pl.ds(off, CH)]], bufs.at[b], gsem.at[b])

        def sc_start(ci, b):
            segbase = s * HPW + jnp.bitwise_and(ci * NPC, HPW - 1)
            for l in range(CH // 16):
                scat[b, pl.ds(l * 16, 16)] = (
                    jnp.zeros((16,), jnp.int32) + (segbase + l * 16 // DEG))
            pltpu.async_copy(bufs.at[b], acc.at[scat.at[b]], ssem.at[b],
                             add=True)

        def sc_wait(b):
            pltpu.make_async_copy(bufs.at[b], acc.at[scat.at[b]],
                                  ssem.at[b]).wait()

        for b in range(NBUF):
            gcopy(b, b).start()
        pltpu.make_async_copy(zeros_h, acc.at[pl.ds(s * HPW, HPW)],
                              zsem).wait()

        def body(j, carry):
            for b in range(NBUF):
                ci = NBUF * j + b
                gcopy(ci, b).wait()
                sc_start(ci, b)
            for b in range(NBUF):
                sc_wait(b)

                @pl.when(j < NGRP - 1)
                def _():
                    gcopy(NBUF * (j + 1) + b, b).start()

            @pl.when(j == NGRP // 2 - 1)
            def _():
                pltpu.sync_copy(acc.at[pl.ds(s * HPW, HPW)],
                                nsum_o.at[pl.ds(pl.multiple_of(base, CH),
                                                HPW)])
                pltpu.sync_copy(zeros_h, acc.at[pl.ds(s * HPW, HPW)])
            return carry

        lax.fori_loop(0, NGRP, body, 0)

        def sget(kk, b):
            return pltpu.make_async_copy(
                table_h.at[nidx.at[pl.ds(kk * CH, CH)]], bufs.at[b], gsem.at[b])

        for kk in range(BPW // CH):
            sget(kk, kk % NBUF).start()
        for kk in range(BPW // CH):
            sget(kk, kk % NBUF).wait()
            dst = pl.multiple_of(base + kk * CH, CH)
            pltpu.sync_copy(bufs.at[kk % NBUF], self_o.at[pl.ds(dst, CH)])

        pltpu.sync_copy(acc.at[pl.ds(s * HPW, HPW)],
                        nsum_o.at[pl.ds(pl.multiple_of(base + HPW, CH), HPW)])

    return k(table, neigh_flat, nodes, zeros)


def _tc_dense(self_feats, nsum, W1, b1, gamma, beta):
    def body(x_ref, n_ref, w_ref, b_ref, g_ref, bb_ref, o_ref):
        x = x_ref[...]
        n = n_ref[...] * (1.0 / DEG)
        w = w_ref[...]
        lin = lax.dot_general(x, w[:, :D], (((1,), (1,)), ((), ())),
                              preferred_element_type=jnp.float32)
        lin = lin + lax.dot_general(n, w[:, D:], (((1,), (1,)), ((), ())),
                                    preferred_element_type=jnp.float32)
        lin = lin + b_ref[...]
        mu = jnp.mean(lin, axis=0, keepdims=True)
        xc = lin - mu
        var = jnp.mean(xc * xc, axis=0, keepdims=True)
        y = xc * lax.rsqrt(var + 1e-5) * g_ref[...] + bb_ref[...]
        o_ref[...] = jnp.maximum(y, 0.0)

    return pl.pallas_call(
        body,
        out_shape=jax.ShapeDtypeStruct((B, D), jnp.float32),
    )(self_feats, nsum, W1,
      b1.reshape(1, D), gamma.reshape(1, D), beta.reshape(1, D))


def kernel(nodes, neighbors, emb_table, W1, b1, gamma, beta):
    zeros = jnp.zeros((HPW, D), jnp.float32)
    self_feats, nsum = _sc_gather(emb_table, neighbors.reshape(-1), nodes, zeros)
    return _tc_dense(self_feats, nsum, W1, b1, gamma, beta)
